# baseline (speedup 1.0000x reference)
import functools

import jax
import jax.numpy as jnp
from jax import lax
from jax.experimental import pallas as pl
from jax.experimental.pallas import tpu as pltpu
from jax.experimental.pallas import tpu_sc as plsc

BATCH = 4
SEQ = 4096
TOTAL = BATCH * SEQ
DIM = 1024
PAD = 1
L = 16

NC = 2
NS = 16
NW = NC * NS
SPAN = SEQ // NW
CHUNK = 32
NCHUNK = SPAN // CHUNK
NB = 3

_sc_mesh = plsc.VectorSubcoreMesh(core_axis_name="c", subcore_axis_name="s")


@functools.partial(
    pl.kernel,
    mesh=_sc_mesh,
    out_type=jax.ShapeDtypeStruct((TOTAL, DIM), jnp.float32),
    scratch_types=[
        pltpu.VMEM((BATCH, SEQ), jnp.int32),
        pltpu.VMEM((BATCH * NCHUNK, CHUNK), jnp.int32),
        pltpu.VMEM((BATCH, L), jnp.int32),
        pltpu.VMEM((CHUNK, DIM), jnp.float32),
        pltpu.VMEM((CHUNK, DIM), jnp.float32),
        pltpu.VMEM((CHUNK, DIM), jnp.float32),
        pltpu.SemaphoreType.DMA,
        pltpu.SemaphoreType.DMA,
        pltpu.SemaphoreType.DMA,
        pltpu.SemaphoreType.DMA,
        pltpu.SemaphoreType.DMA,
        pltpu.SemaphoreType.DMA,
        pltpu.SemaphoreType.DMA,
    ],
)
def _sc_embed(ids_hbm, table_hbm, out_hbm, row_v, idx_v, stat_v,
              b0, b1, b2, gs0, gs1, gs2, ps0, ps1, ps2, ps3):
    bufs = (b0, b1, b2)
    gsems = (gs0, gs1, gs2)
    psems = (ps0, ps1, ps2, ps3)
    w = lax.axis_index("c") * NS + lax.axis_index("s")
    span = w * SPAN

    pltpu.sync_copy(ids_hbm, row_v)

    iota = lax.iota(jnp.int32, L)
    last = jnp.full((L,), L - 1, jnp.int32)

    _gdn = lax.GatherDimensionNumbers(
        offset_dims=(), collapsed_slice_dims=(0,), start_index_map=(0,))

    def _take(v, i):
        return lax.gather(
            v, i[:, None], _gdn, (1,),
            mode=lax.GatherScatterMode.PROMISE_IN_BOUNDS)

    for b in range(BATCH):
        def _count_body(j, acc, b=b):
            v = row_v[b, pl.ds(j * L, L)]
            m = jnp.minimum(jnp.abs(v - PAD), 1)
            sel = jnp.minimum(jnp.maximum(w * (SPAN // L) - j, 0), 1)
            return acc + m * sel

        acc = lax.fori_loop(0, SEQ // L, _count_body,
                            jnp.zeros((L,), jnp.int32))
        prefix_v = acc
        for sh in (1, 2, 4, 8):
            prefix_v = prefix_v + _take(prefix_v, iota ^ sh)

        diff = jnp.zeros((L,), jnp.int32)
        for j in range(SPAN // L):
            v = row_v[b, pl.ds(span + j * L, L)]
            m = jnp.minimum(jnp.abs(v - PAD), 1)
            s = m
            for sh in (1, 2, 4, 8):
                keep = jnp.minimum(jnp.maximum(iota - (sh - 1), 0), 1)
                s = s + _take(s, jnp.maximum(iota - sh, 0)) * keep
            idx = (prefix_v + s) * m + PAD
            c = j * L // CHUNK
            off = (j * L) % CHUNK
            idx_v[b * NCHUNK + c, pl.ds(off, L)] = idx
            if b > 0:
                diff = diff + jnp.abs(idx - idx_v[c, pl.ds(off, L)])
            prefix_v = prefix_v + _take(s, last)
        if b > 0:
            for sh in (1, 2, 4, 8):
                diff = diff + _take(diff, iota ^ sh)
            stat_v[b, pl.ds(0, L)] = diff

    def _get(c, slot, idxrow):
        return pltpu.async_copy(
            table_hbm.at[idx_v.at[idxrow]], bufs[slot], gsems[slot])

    def _put(c, slot, b):
        return pltpu.async_copy(
            bufs[slot],
            out_hbm.at[pl.ds(b * SEQ + span + c * CHUNK, CHUNK)],
            psems[b])

    puts = [[None] * BATCH for _ in range(NCHUNK)]
    gets = [None] * NCHUNK
    gets[0] = _get(0, 0, 0)
    gets[1] = _get(1, 1, 1)
    gets[2] = _get(2, 2, 2)
    gets[0].wait()
    for b in range(BATCH):
        puts[0][b] = _put(0, 0, b)
    gets[1].wait()
    for b in range(BATCH):
        puts[1][b] = _put(1, 1, b)
    for b in range(BATCH):
        puts[0][b].wait()
    gets[3] = _get(3, 0, 3)
    gets[2].wait()
    for b in range(BATCH):
        puts[2][b] = _put(2, 2, b)
    gets[3].wait()
    for b in range(BATCH):
        puts[3][b] = _put(3, 0, b)
    for c in range(1, NCHUNK):
        for b in range(BATCH):
            puts[c][b].wait()

    for b in range(1, BATCH):
        neq = stat_v[b, pl.ds(0, L)][0] != 0

        @pl.when(neq)
        def _redo(b=b):
            for c in range(NCHUNK):
                g = _get(c, 0, b * NCHUNK + c)
                g.wait()
                p = _put(c, 0, b)
                p.wait()


def kernel(input_ids, weights):
    out = _sc_embed(input_ids, weights)
    return out.reshape(BATCH, SEQ, weights.shape[-1])


# speedup vs baseline: 1.0345x; 1.0345x over previous
import functools

import jax
import jax.numpy as jnp
from jax import lax
from jax.experimental import pallas as pl
from jax.experimental.pallas import tpu as pltpu
from jax.experimental.pallas import tpu_sc as plsc

BATCH = 4
SEQ = 4096
TOTAL = BATCH * SEQ
DIM = 1024
PAD = 1
L = 16

NC = 2
NS = 16
NW = NC * NS
SPAN = SEQ // NW
CHUNK = 32
NCHUNK = SPAN // CHUNK
NB = 3

_sc_mesh = plsc.VectorSubcoreMesh(core_axis_name="c", subcore_axis_name="s")


@functools.partial(
    pl.kernel,
    mesh=_sc_mesh,
    out_type=jax.ShapeDtypeStruct((TOTAL, DIM), jnp.float32),
    scratch_types=[
        pltpu.VMEM((BATCH, SEQ), jnp.int32),
        pltpu.VMEM((BATCH * NCHUNK, CHUNK), jnp.int32),
        pltpu.VMEM((BATCH, L), jnp.int32),
        pltpu.VMEM((2, L), jnp.int32),
        pltpu.VMEM((NS, 2, L), jnp.int32),
        pltpu.VMEM_SHARED((NS, 2, L), jnp.int32),
        pltpu.VMEM((CHUNK, DIM), jnp.float32),
        pltpu.VMEM((CHUNK, DIM), jnp.float32),
        pltpu.VMEM((CHUNK, DIM), jnp.float32),
        pltpu.SemaphoreType.DMA,
        pltpu.SemaphoreType.DMA,
        pltpu.SemaphoreType.DMA,
        pltpu.SemaphoreType.DMA,
        pltpu.SemaphoreType.DMA,
        pltpu.SemaphoreType.DMA,
        pltpu.SemaphoreType.DMA,
    ],
)
def _sc_embed(ids_hbm, table_hbm, out_hbm, row_v, idx_v, stat_v,
              ssum_v, all_v, shared_v,
              b0, b1, b2, gs0, gs1, gs2, ps0, ps1, ps2, ps3):
    bufs = (b0, b1, b2)
    gsems = (gs0, gs1, gs2)
    psems = (ps0, ps1, ps2, ps3)
    w = lax.axis_index("c") * NS + lax.axis_index("s")
    span = w * SPAN

    pltpu.sync_copy(ids_hbm, row_v)

    iota = lax.iota(jnp.int32, L)
    last = jnp.full((L,), L - 1, jnp.int32)

    _gdn = lax.GatherDimensionNumbers(
        offset_dims=(), collapsed_slice_dims=(0,), start_index_map=(0,))

    def _take(v, i):
        return lax.gather(
            v, i[:, None], _gdn, (1,),
            mode=lax.GatherScatterMode.PROMISE_IN_BOUNDS)

    s_id = lax.axis_index("s")
    for p in range(2):
        comb = jnp.zeros((L,), jnp.int32)
        for b in range(BATCH):
            accv = jnp.zeros((L,), jnp.int32)
            for g in range(SPAN // L):
                v = row_v[b, pl.ds((2 * s_id + p) * SPAN + g * L, L)]
                accv = accv + jnp.minimum(jnp.abs(v - PAD), 1)
            for sh in (1, 2, 4, 8):
                accv = accv + _take(accv, iota ^ sh)
            lane_b = jnp.minimum(jnp.maximum(1 - jnp.abs(iota - b), 0), 1)
            comb = comb + accv * lane_b
        ssum_v[p, pl.ds(0, L)] = comb
    pltpu.sync_copy(ssum_v, shared_v.at[s_id])
    plsc.subcore_barrier()
    pltpu.sync_copy(shared_v, all_v)
    pacc = jnp.zeros((L,), jnp.int32)
    for t in range(NS):
        for p in range(2):
            sel = jnp.minimum(jnp.maximum(w - (2 * t + p), 0), 1)
            pacc = pacc + all_v[t, p, pl.ds(0, L)] * sel

    for b in range(BATCH):
        prefix_v = _take(pacc, jnp.full((L,), b, jnp.int32))

        diff = jnp.zeros((L,), jnp.int32)
        for j in range(SPAN // L):
            v = row_v[b, pl.ds(span + j * L, L)]
            m = jnp.minimum(jnp.abs(v - PAD), 1)
            s = m
            for sh in (1, 2, 4, 8):
                keep = jnp.minimum(jnp.maximum(iota - (sh - 1), 0), 1)
                s = s + _take(s, jnp.maximum(iota - sh, 0)) * keep
            idx = (prefix_v + s) * m + PAD
            c = j * L // CHUNK
            off = (j * L) % CHUNK
            idx_v[b * NCHUNK + c, pl.ds(off, L)] = idx
            if b > 0:
                diff = diff + jnp.abs(idx - idx_v[c, pl.ds(off, L)])
            prefix_v = prefix_v + _take(s, last)
        if b > 0:
            for sh in (1, 2, 4, 8):
                diff = diff + _take(diff, iota ^ sh)
            stat_v[b, pl.ds(0, L)] = diff

    def _get(c, slot, idxrow):
        return pltpu.async_copy(
            table_hbm.at[idx_v.at[idxrow]], bufs[slot], gsems[slot])

    def _put(c, slot, b):
        return pltpu.async_copy(
            bufs[slot],
            out_hbm.at[pl.ds(b * SEQ + span + c * CHUNK, CHUNK)],
            psems[b])

    puts = [[None] * BATCH for _ in range(NCHUNK)]
    gets = [None] * NCHUNK
    gets[0] = _get(0, 0, 0)
    gets[1] = _get(1, 1, 1)
    gets[2] = _get(2, 2, 2)
    gets[0].wait()
    for b in range(BATCH):
        puts[0][b] = _put(0, 0, b)
    gets[1].wait()
    for b in range(BATCH):
        puts[1][b] = _put(1, 1, b)
    for b in range(BATCH):
        puts[0][b].wait()
    gets[3] = _get(3, 0, 3)
    gets[2].wait()
    for b in range(BATCH):
        puts[2][b] = _put(2, 2, b)
    gets[3].wait()
    for b in range(BATCH):
        puts[3][b] = _put(3, 0, b)
    for c in range(1, NCHUNK):
        for b in range(BATCH):
            puts[c][b].wait()

    for b in range(1, BATCH):
        neq = stat_v[b, pl.ds(0, L)][0] != 0

        @pl.when(neq)
        def _redo(b=b):
            for c in range(NCHUNK):
                g = _get(c, 0, b * NCHUNK + c)
                g.wait()
                p = _put(c, 0, b)
                p.wait()


def kernel(input_ids, weights):
    out = _sc_embed(input_ids, weights)
    return out.reshape(BATCH, SEQ, weights.shape[-1])


# speedup vs baseline: 1.1264x; 1.0889x over previous
import functools

import jax
import jax.numpy as jnp
from jax import lax
from jax.experimental import pallas as pl
from jax.experimental.pallas import tpu as pltpu
from jax.experimental.pallas import tpu_sc as plsc

BATCH = 4
SEQ = 4096
TOTAL = BATCH * SEQ
DIM = 1024
PAD = 1
L = 16

NC = 2
NS = 16
NW = NC * NS
BPW = TOTAL // NW
WPR = SEQ // BPW
CHUNK = 32
NCHUNK = BPW // CHUNK
NB = 3

_sc_mesh = plsc.VectorSubcoreMesh(core_axis_name="c", subcore_axis_name="s")


@functools.partial(
    pl.kernel,
    mesh=_sc_mesh,
    out_type=jax.ShapeDtypeStruct((TOTAL, DIM), jnp.float32),
    scratch_types=[
        pltpu.VMEM((SEQ,), jnp.int32),
        pltpu.VMEM((NCHUNK, CHUNK), jnp.int32),
        pltpu.VMEM((CHUNK, DIM), jnp.float32),
        pltpu.VMEM((CHUNK, DIM), jnp.float32),
        pltpu.VMEM((CHUNK, DIM), jnp.float32),
        pltpu.SemaphoreType.DMA,
        pltpu.SemaphoreType.DMA,
        pltpu.SemaphoreType.DMA,
        pltpu.SemaphoreType.DMA,
        pltpu.SemaphoreType.DMA,
        pltpu.SemaphoreType.DMA,
    ],
)
def _sc_embed(ids_hbm, table_hbm, out_hbm, row_v, idx_v,
              b0, b1, b2, gs0, gs1, gs2, ps0, ps1, ps2):
    bufs = (b0, b1, b2)
    gsems = (gs0, gs1, gs2)
    psems = (ps0, ps1, ps2)
    wid = lax.axis_index("c") * NS + lax.axis_index("s")
    row = wid // WPR
    k = wid % WPR
    base = wid * BPW

    pltpu.sync_copy(ids_hbm.at[row], row_v)

    iota = lax.iota(jnp.int32, L)
    last = jnp.full((L,), L - 1, jnp.int32)

    _gdn = lax.GatherDimensionNumbers(
        offset_dims=(), collapsed_slice_dims=(0,), start_index_map=(0,))

    def _take(v, i):
        return lax.gather(
            v, i[:, None], _gdn, (1,),
            mode=lax.GatherScatterMode.PROMISE_IN_BOUNDS)

    def _count_body(j, acc):
        v = row_v[pl.ds(j * L, L)]
        return acc + jnp.minimum(jnp.abs(v - PAD), 1)

    acc = lax.fori_loop(0, k * (BPW // L), _count_body,
                        jnp.zeros((L,), jnp.int32))
    prefix_v = acc
    for sh in (1, 2, 4, 8):
        prefix_v = prefix_v + _take(prefix_v, iota ^ sh)

    span = k * BPW
    for j in range(BPW // L):
        v = row_v[pl.ds(span + j * L, L)]
        m = jnp.minimum(jnp.abs(v - PAD), 1)
        s = m
        for sh in (1, 2, 4, 8):
            keep = jnp.minimum(jnp.maximum(iota - (sh - 1), 0), 1)
            s = s + _take(s, jnp.maximum(iota - sh, 0)) * keep
        idx_v[j * L // CHUNK, pl.ds((j * L) % CHUNK, L)] = (prefix_v + s) * m + PAD
        prefix_v = prefix_v + _take(s, last)

    gets = [None] * NCHUNK
    puts = [None] * NCHUNK

    def _get(c):
        b = c % NB
        return pltpu.async_copy(
            table_hbm.at[idx_v.at[c]], bufs[b], gsems[b]
        )

    def _put(c):
        b = c % NB
        return pltpu.async_copy(
            bufs[b], out_hbm.at[pl.ds(base + c * CHUNK, CHUNK)], psems[b]
        )

    gets[0] = _get(0)
    gets[1] = _get(1)
    for c in range(NCHUNK):
        gets[c].wait()
        puts[c] = _put(c)
        if c + 2 < NCHUNK:
            if c >= 1:
                puts[c - 1].wait()
            gets[c + 2] = _get(c + 2)
    for c in range(NCHUNK - NB, NCHUNK):
        puts[c].wait()


def kernel(input_ids, weights):
    out = _sc_embed(input_ids, weights)
    return out.reshape(BATCH, SEQ, weights.shape[-1])
